# baseline (speedup 1.0000x reference)
import dataclasses
import functools

import jax
import jax.numpy as jnp
from jax import lax
from jax.experimental import pallas as pl
from jax.experimental.pallas import tpu as pltpu
from jax.experimental.pallas import tpu_sc as plsc

MAX_SCORE = 2.0
P = 70
PP = 80
L = 16
NC = 2
NS = 16
NW = NC * NS


@functools.lru_cache(maxsize=None)
def _make_sc_segment_sums(N, D, chunk):
    tokw = N // NW
    npairs = tokw // (2 * chunk)
    mesh = plsc.VectorSubcoreMesh(core_axis_name="c", subcore_axis_name="s")
    cp = pltpu.CompilerParams()
    if "needs_layout_passes" in pltpu.CompilerParams.__dataclass_fields__:
        cp = dataclasses.replace(cp, needs_layout_passes=False)

    @functools.partial(
        pl.kernel,
        mesh=mesh,
        out_type=[
            jax.ShapeDtypeStruct((NW * PP * D,), jnp.float32),
            jax.ShapeDtypeStruct((NW, PP), jnp.float32),
            jax.ShapeDtypeStruct((NW, PP), jnp.float32),
        ],
        scratch_types=[
            pltpu.VMEM((chunk * D,), jnp.float32),
            pltpu.VMEM((chunk * D,), jnp.float32),
            pltpu.VMEM((tokw,), jnp.int32),
            pltpu.VMEM((tokw,), jnp.float32),
            pltpu.VMEM((PP * D,), jnp.float32),
            pltpu.VMEM((L * PP,), jnp.float32),
            pltpu.VMEM((L * PP,), jnp.float32),
            pltpu.VMEM((PP,), jnp.float32),
            pltpu.VMEM((PP,), jnp.float32),
            pltpu.SemaphoreType.DMA,
            pltpu.SemaphoreType.DMA,
        ],
        compiler_params=cp,
    )
    def sc_segment_sums(feat_hbm, phn_hbm, sc_hbm, cent_out, cnt_out, hc_out,
                        buf0, buf1, phn_v, sc_v, acc_v,
                        cnt16_v, hc16_v, cntred_v, hcred_v, sem0, sem1):
        cid = lax.axis_index("c")
        sid = lax.axis_index("s")
        wid = sid * NC + cid
        lane = lax.iota(jnp.int32, L)
        lane_pp = lane * PP
        zeros = jnp.zeros((L,), jnp.float32)
        base = wid * tokw

        @pl.loop(0, PP * D, step=L)
        def _(i):
            acc_v[pl.ds(i, L)] = zeros

        @pl.loop(0, L * PP, step=L)
        def _(i):
            cnt16_v[pl.ds(i, L)] = zeros
            hc16_v[pl.ds(i, L)] = zeros

        pltpu.sync_copy(phn_hbm.at[pl.ds(base, tokw)], phn_v)
        pltpu.sync_copy(sc_hbm.at[pl.ds(base, tokw)], sc_v)

        @pl.loop(0, tokw, step=L)
        def _(i):
            idxp = phn_v[pl.ds(i, L)]
            s = sc_v[pl.ds(i, L)]
            valid = s >= 0.0
            vf = jnp.where(valid, 1.0, 0.0)
            is2 = jnp.where(valid & (s == MAX_SCORE), 1.0, 0.0)
            plsc.addupdate_scatter(cnt16_v, [lane_pp + idxp], vf)
            plsc.addupdate_scatter(hc16_v, [lane_pp + idxp], is2)

        def feat_copy(ci, buf, sem):
            return pltpu.make_async_copy(
                feat_hbm.at[pl.ds((base + ci * chunk) * D, chunk * D)],
                buf, sem)

        def process(coff, buf):
            @pl.loop(0, chunk, step=2 * L)
            def _(g):
                idxp0 = phn_v[pl.ds(coff + g, L)]
                s0 = sc_v[pl.ds(coff + g, L)]
                idxp1 = phn_v[pl.ds(coff + g + L, L)]
                s1 = sc_v[pl.ds(coff + g + L, L)]
                valid0 = s0 >= 0.0
                valid1 = s1 >= 0.0
                tokbase0 = (g + lane) * D
                tokbase1 = tokbase0 + L * D
                phnbase0 = idxp0 * D
                phnbase1 = idxp1 * D

                @plsc.parallel_loop(0, D, 1, unroll=4, carry=lane)
                def _(c, col):
                    vals0 = plsc.load_gather(buf, [tokbase0 + col])
                    plsc.addupdate_scatter(acc_v, [phnbase0 + col], vals0,
                                           mask=valid0)
                    vals1 = plsc.load_gather(buf, [tokbase1 + col])
                    plsc.addupdate_scatter(acc_v, [phnbase1 + col], vals1,
                                           mask=valid1)
                    return (col + 1) & (D - 1)

        feat_copy(0, buf0, sem0).start()

        @pl.loop(0, npairs)
        def _(i):
            ci = i * 2
            feat_copy(ci, buf0, sem0).wait()
            feat_copy(ci + 1, buf1, sem1).start()
            process(ci * chunk, buf0)
            feat_copy(ci + 1, buf1, sem1).wait()

            @pl.when(i < npairs - 1)
            def _():
                feat_copy(ci + 2, buf0, sem0).start()

            process((ci + 1) * chunk, buf1)

        for j in range(PP // L):
            v = zeros
            h = zeros
            for r in range(L):
                v = v + cnt16_v[pl.ds(r * PP + j * L, L)]
                h = h + hc16_v[pl.ds(r * PP + j * L, L)]
            cntred_v[pl.ds(j * L, L)] = v
            hcred_v[pl.ds(j * L, L)] = h

        pltpu.sync_copy(acc_v, cent_out.at[pl.ds(wid * (PP * D), PP * D)])
        pltpu.sync_copy(cntred_v, cnt_out.at[wid])
        pltpu.sync_copy(hcred_v, hc_out.at[wid])

    return sc_segment_sums


_ENC = 1024.0


def _tc_body(NB, TB, D, lam_ref, feat_ref, pk_ref, cent_ref, cnt_ref,
             hc_ref, out_ref, p_scr, cc_scr, smem):
    i = pl.program_id(0)
    f32 = jnp.float32
    dotp = dict(preferred_element_type=f32, precision=lax.Precision.HIGHEST)

    @pl.when(i == 0)
    def _():
        cnt = cnt_ref[...]
        hc = hc_ref[...]
        csum = cent_ref[pl.ds(0, PP), :]
        for w in range(1, NW):
            csum = csum + cent_ref[pl.ds(w * PP, PP), :]
        ones_c = jnp.ones((NW, 1), f32)
        ones_r = jnp.ones((1, NW), f32)
        cn_col = lax.dot_general(cnt, ones_c, (((0,), (0,)), ((), ())), **dotp)
        hc_col = lax.dot_general(hc, ones_c, (((0,), (0,)), ((), ())), **dotp)
        cn_row = lax.dot_general(ones_r, cnt, (((1,), (0,)), ((), ())), **dotp)
        hc_row = lax.dot_general(ones_r, hc, (((1,), (0,)), ((), ())), **dotp)
        presc = hc_col > 0.0
        presr = hc_row > 0.0
        counts_col = jnp.where(presc, cn_col, 0.0)
        counts_row = jnp.where(presr, cn_row, 0.0)
        center = csum / jnp.maximum(counts_col, 1.0)
        cn2 = jnp.sum(center * center, axis=1, keepdims=True)
        inv = 1.0 / jnp.maximum(jnp.sqrt(cn2), 1e-12)
        pmat = center * inv
        cc_col = cn2 * inv * inv
        pg = lax.dot_general(pmat, pmat, (((1,), (1,)), ((), ())), **dotp)
        r0 = lax.broadcasted_iota(jnp.int32, (PP, PP), 0)
        r1 = lax.broadcasted_iota(jnp.int32, (PP, PP), 1)
        eye = jnp.where(r0 == r1, 1.0, 0.0)
        cc_c = jnp.sum(pg * eye, axis=1, keepdims=True)
        cc_r = jnp.sum(pg * eye, axis=0, keepdims=True)
        dist = jnp.sqrt(jnp.maximum(cc_c + cc_r - 2.0 * pg, 1e-12))
        pairm = jnp.where((r0 < r1) & presc & presr, 1.0, 0.0)
        divden = jnp.sum(pairm)
        smem[3] = jnp.sum(dist * pairm) / jnp.maximum(divden, 1.0)
        smem[4] = jnp.sum(counts_row)
        smem[5] = jnp.sum(jnp.where(presr, 1.0, 0.0))
        smem[0] = 0.0
        smem[1] = 0.0
        smem[2] = 0.0
        p_scr[...] = pmat.astype(jnp.bfloat16)
        presf = jnp.where(presc, 1.0, 0.0)
        cc_scr[...] = jnp.where(presc, cc_col, 0.0) + _ENC * presf

    fb = feat_ref[...].astype(jnp.bfloat16)
    gt = lax.dot_general(p_scr[...], fb, (((1,), (1,)), ((), ())),
                         preferred_element_type=f32)
    sqt = lax.dot_general(
        jnp.ones((1, D), jnp.bfloat16), fb * fb, (((1,), (1,)), ((), ())),
        preferred_element_type=f32)
    pk = pk_ref[0]
    phnrow = pk & 127
    srow = lax.shift_right_logical(pk, 7).astype(f32)
    iota_p = lax.broadcasted_iota(jnp.int32, (PP, TB), 0)
    oh = jnp.where(phnrow == iota_p, 1.0, 0.0)
    fninv = 1.0 / jnp.maximum(jnp.sqrt(sqt), 1e-12)
    fnfn = sqt * fninv * fninv
    m = cc_scr[...] - (2.0 * fninv) * gt
    sel2 = jnp.sum(oh * m, axis=0, keepdims=True)
    pres_t = sel2 > 0.5 * _ENC
    diff = fnfn + sel2 - _ENC * jnp.where(pres_t, 1.0, 0.0)
    nz = pres_t & (diff > 0.0)
    nzf = jnp.where(nz, 1.0, 0.0)
    contrib = jnp.sqrt(jnp.maximum(diff, 0.0)) * nzf
    smem[0] += jnp.sum(contrib * (MAX_SCORE - srow))
    smem[1] += jnp.sum(contrib)
    smem[2] += jnp.sum(nzf)

    s2 = smem[2]
    tight = (smem[0] + lam_ref[2] * smem[1]) / jnp.maximum(s2, 1.0)
    loss = jnp.where(s2 > 0.0,
                     lam_ref[1] * tight - lam_ref[0] * smem[3],
                     -lam_ref[0] * smem[3])
    out_ref[0] = jnp.where((smem[4] > 0.0) & (smem[5] >= 2.0), loss, 0.0)


@functools.lru_cache(maxsize=None)
def _make_tc_loss(N, D, NB, TB, interpret=False):
    body = functools.partial(_tc_body, NB, TB, D)
    return pl.pallas_call(
        body,
        grid=(NB,),
        in_specs=[
            pl.BlockSpec(memory_space=pltpu.SMEM),
            pl.BlockSpec((TB, D), lambda i: (i, 0)),
            pl.BlockSpec((1, 1, TB), lambda i: (i, 0, 0)),
            pl.BlockSpec((NW * PP, D), lambda i: (0, 0)),
            pl.BlockSpec((NW, PP), lambda i: (0, 0)),
            pl.BlockSpec((NW, PP), lambda i: (0, 0)),
        ],
        out_specs=pl.BlockSpec(memory_space=pltpu.SMEM),
        out_shape=jax.ShapeDtypeStruct((1,), jnp.float32),
        scratch_shapes=[
            pltpu.VMEM((PP, D), jnp.bfloat16),
            pltpu.VMEM((PP, 1), jnp.float32),
            pltpu.SMEM((8,), jnp.float32),
        ],
        interpret=interpret,
    )


def kernel(features, scores, phn_ids, lambda_d=0.5, lambda_t=0.1, margin=1.0):
    B, T, D = features.shape
    N = B * T
    feat2 = features.reshape(N, D)
    featf = features.reshape(-1)
    phn = phn_ids.reshape(-1).astype(jnp.int32)
    sc = scores.reshape(-1).astype(jnp.float32)

    cent_parts, cnt_parts, hc_parts = _make_sc_segment_sums(N, D, 128)(
        featf, phn, sc)
    cent2 = cent_parts.reshape(NW * PP, D)

    NB, TB = 8, N // 8
    lam = jnp.stack([jnp.asarray(lambda_d, jnp.float32),
                     jnp.asarray(lambda_t, jnp.float32),
                     jnp.asarray(margin, jnp.float32)])
    packed = phn | (sc.astype(jnp.int32) << 7)
    out = _make_tc_loss(N, D, NB, TB)(
        lam, feat2, packed.reshape(NB, 1, TB),
        cent2, cnt_parts, hc_parts)
    return out[0]


# speedup vs baseline: 1.0253x; 1.0253x over previous
import dataclasses
import functools

import jax
import jax.numpy as jnp
from jax import lax
from jax.experimental import pallas as pl
from jax.experimental.pallas import tpu as pltpu
from jax.experimental.pallas import tpu_sc as plsc

MAX_SCORE = 2.0
P = 70
PP = 80
L = 16
NC = 2
NS = 16
NW = NC * NS


@functools.lru_cache(maxsize=None)
def _make_sc_segment_sums(N, D, chunk):
    tokw = N // NW
    npairs = tokw // (2 * chunk)
    mesh = plsc.VectorSubcoreMesh(core_axis_name="c", subcore_axis_name="s")
    cp = pltpu.CompilerParams()
    if "needs_layout_passes" in pltpu.CompilerParams.__dataclass_fields__:
        cp = dataclasses.replace(cp, needs_layout_passes=False)

    @functools.partial(
        pl.kernel,
        mesh=mesh,
        out_type=[
            jax.ShapeDtypeStruct((NW * PP * D,), jnp.float32),
            jax.ShapeDtypeStruct((NW, PP), jnp.float32),
            jax.ShapeDtypeStruct((NW, PP), jnp.float32),
        ],
        scratch_types=[
            pltpu.VMEM((chunk * D,), jnp.float32),
            pltpu.VMEM((chunk * D,), jnp.float32),
            pltpu.VMEM((tokw,), jnp.int32),
            pltpu.VMEM((tokw,), jnp.float32),
            pltpu.VMEM((PP * D,), jnp.float32),
            pltpu.VMEM((L * PP,), jnp.float32),
            pltpu.VMEM((L * PP,), jnp.float32),
            pltpu.VMEM((PP,), jnp.float32),
            pltpu.VMEM((PP,), jnp.float32),
            pltpu.SemaphoreType.DMA,
            pltpu.SemaphoreType.DMA,
        ],
        compiler_params=cp,
    )
    def sc_segment_sums(feat_hbm, phn_hbm, sc_hbm, cent_out, cnt_out, hc_out,
                        buf0, buf1, phn_v, sc_v, acc_v,
                        cnt16_v, hc16_v, cntred_v, hcred_v, sem0, sem1):
        cid = lax.axis_index("c")
        sid = lax.axis_index("s")
        wid = sid * NC + cid
        lane = lax.iota(jnp.int32, L)
        lane_pp = lane * PP
        zeros = jnp.zeros((L,), jnp.float32)
        base = wid * tokw

        @pl.loop(0, PP * D, step=L)
        def _(i):
            acc_v[pl.ds(i, L)] = zeros

        @pl.loop(0, L * PP, step=L)
        def _(i):
            cnt16_v[pl.ds(i, L)] = zeros
            hc16_v[pl.ds(i, L)] = zeros

        pltpu.sync_copy(phn_hbm.at[pl.ds(base, tokw)], phn_v)
        pltpu.sync_copy(sc_hbm.at[pl.ds(base, tokw)], sc_v)

        @pl.loop(0, tokw, step=L)
        def _(i):
            idxp = phn_v[pl.ds(i, L)]
            s = sc_v[pl.ds(i, L)]
            valid = s >= 0.0
            vf = jnp.where(valid, 1.0, 0.0)
            is2 = jnp.where(valid & (s == MAX_SCORE), 1.0, 0.0)
            plsc.addupdate_scatter(cnt16_v, [lane_pp + idxp], vf)
            plsc.addupdate_scatter(hc16_v, [lane_pp + idxp], is2)

        def feat_copy(ci, buf, sem):
            return pltpu.make_async_copy(
                feat_hbm.at[pl.ds((base + ci * chunk) * D, chunk * D)],
                buf, sem)

        def process(coff, buf):
            @pl.loop(0, chunk, step=2 * L)
            def _(g):
                idxp0 = phn_v[pl.ds(coff + g, L)]
                s0 = sc_v[pl.ds(coff + g, L)]
                idxp1 = phn_v[pl.ds(coff + g + L, L)]
                s1 = sc_v[pl.ds(coff + g + L, L)]
                valid0 = s0 >= 0.0
                valid1 = s1 >= 0.0
                tokbase0 = (g + lane) * D
                tokbase1 = tokbase0 + L * D
                phnbase0 = idxp0 * D
                phnbase1 = idxp1 * D

                @plsc.parallel_loop(0, D, 1, unroll=4, carry=lane)
                def _(c, col):
                    vals0 = plsc.load_gather(buf, [tokbase0 + col])
                    plsc.addupdate_scatter(acc_v, [phnbase0 + col], vals0,
                                           mask=valid0)
                    vals1 = plsc.load_gather(buf, [tokbase1 + col])
                    plsc.addupdate_scatter(acc_v, [phnbase1 + col], vals1,
                                           mask=valid1)
                    return (col + 1) & (D - 1)

        feat_copy(0, buf0, sem0).start()

        @pl.loop(0, npairs)
        def _(i):
            ci = i * 2
            feat_copy(ci, buf0, sem0).wait()
            feat_copy(ci + 1, buf1, sem1).start()
            process(ci * chunk, buf0)
            feat_copy(ci + 1, buf1, sem1).wait()

            @pl.when(i < npairs - 1)
            def _():
                feat_copy(ci + 2, buf0, sem0).start()

            process((ci + 1) * chunk, buf1)

        for j in range(PP // L):
            v = zeros
            h = zeros
            for r in range(L):
                v = v + cnt16_v[pl.ds(r * PP + j * L, L)]
                h = h + hc16_v[pl.ds(r * PP + j * L, L)]
            cntred_v[pl.ds(j * L, L)] = v
            hcred_v[pl.ds(j * L, L)] = h

        pltpu.sync_copy(acc_v, cent_out.at[pl.ds(wid * (PP * D), PP * D)])
        pltpu.sync_copy(cntred_v, cnt_out.at[wid])
        pltpu.sync_copy(hcred_v, hc_out.at[wid])

    return sc_segment_sums


_ENC = 1024.0


def _tc_body(NB, TB, D, lam_ref, feat_ref, pk_ref, cent_ref, cnt_ref,
             hc_ref, out_ref, p_scr, cc_scr, smem):
    i = pl.program_id(0)
    f32 = jnp.float32
    dotp = dict(preferred_element_type=f32, precision=lax.Precision.HIGHEST)

    @pl.when(i == 0)
    def _():
        cnt = cnt_ref[...]
        hc = hc_ref[...]
        csum = cent_ref[pl.ds(0, PP), :]
        for w in range(1, NW):
            csum = csum + cent_ref[pl.ds(w * PP, PP), :]
        ones_c = jnp.ones((NW, 1), f32)
        ones_r = jnp.ones((1, NW), f32)
        cn_col = lax.dot_general(cnt, ones_c, (((0,), (0,)), ((), ())), **dotp)
        hc_col = lax.dot_general(hc, ones_c, (((0,), (0,)), ((), ())), **dotp)
        cn_row = lax.dot_general(ones_r, cnt, (((1,), (0,)), ((), ())), **dotp)
        hc_row = lax.dot_general(ones_r, hc, (((1,), (0,)), ((), ())), **dotp)
        presc = hc_col > 0.0
        presr = hc_row > 0.0
        counts_col = jnp.where(presc, cn_col, 0.0)
        counts_row = jnp.where(presr, cn_row, 0.0)
        center = csum / jnp.maximum(counts_col, 1.0)
        cn2 = jnp.sum(center * center, axis=1, keepdims=True)
        inv = 1.0 / jnp.maximum(jnp.sqrt(cn2), 1e-12)
        pmat = center * inv
        cc_col = cn2 * inv * inv
        pg = lax.dot_general(pmat, pmat, (((1,), (1,)), ((), ())), **dotp)
        r0 = lax.broadcasted_iota(jnp.int32, (PP, PP), 0)
        r1 = lax.broadcasted_iota(jnp.int32, (PP, PP), 1)
        eye = jnp.where(r0 == r1, 1.0, 0.0)
        cc_c = jnp.sum(pg * eye, axis=1, keepdims=True)
        cc_r = jnp.sum(pg * eye, axis=0, keepdims=True)
        dist = jnp.sqrt(jnp.maximum(cc_c + cc_r - 2.0 * pg, 1e-12))
        pairm = jnp.where((r0 < r1) & presc & presr, 1.0, 0.0)
        divden = jnp.sum(pairm)
        smem[3] = jnp.sum(dist * pairm) / jnp.maximum(divden, 1.0)
        smem[4] = jnp.sum(counts_row)
        smem[5] = jnp.sum(jnp.where(presr, 1.0, 0.0))
        smem[0] = 0.0
        smem[1] = 0.0
        smem[2] = 0.0
        p_scr[...] = pmat.astype(jnp.bfloat16)
        presf = jnp.where(presc, 1.0, 0.0)
        cc_scr[...] = jnp.where(presc, cc_col, 0.0) + _ENC * presf

    fb = feat_ref[...].astype(jnp.bfloat16)
    gt = lax.dot_general(p_scr[...], fb, (((1,), (1,)), ((), ())),
                         preferred_element_type=f32)
    sqt = lax.dot_general(
        jnp.ones((1, D), jnp.bfloat16), fb * fb, (((1,), (1,)), ((), ())),
        preferred_element_type=f32)
    pk = pk_ref[0]
    phnrow = pk & 127
    srow = lax.shift_right_logical(pk, 7).astype(f32)
    iota_p = lax.broadcasted_iota(jnp.int32, (PP, TB), 0)
    oh = jnp.where(phnrow == iota_p, 1.0, 0.0)
    fninv = 1.0 / jnp.maximum(jnp.sqrt(sqt), 1e-12)
    fnfn = sqt * fninv * fninv
    m = cc_scr[...] - (2.0 * fninv) * gt
    sel2 = jnp.sum(oh * m, axis=0, keepdims=True)
    pres_t = sel2 > 0.5 * _ENC
    diff = fnfn + sel2 - _ENC * jnp.where(pres_t, 1.0, 0.0)
    nz = pres_t & (diff > 0.0)
    nzf = jnp.where(nz, 1.0, 0.0)
    contrib = jnp.sqrt(jnp.maximum(diff, 0.0)) * nzf
    smem[0] += jnp.sum(contrib * (MAX_SCORE - srow))
    smem[1] += jnp.sum(contrib)
    smem[2] += jnp.sum(nzf)

    s2 = smem[2]
    tight = (smem[0] + lam_ref[2] * smem[1]) / jnp.maximum(s2, 1.0)
    loss = jnp.where(s2 > 0.0,
                     lam_ref[1] * tight - lam_ref[0] * smem[3],
                     -lam_ref[0] * smem[3])
    out_ref[0] = jnp.where((smem[4] > 0.0) & (smem[5] >= 2.0), loss, 0.0)


@functools.lru_cache(maxsize=None)
def _make_tc_loss(N, D, NB, TB, interpret=False):
    body = functools.partial(_tc_body, NB, TB, D)
    return pl.pallas_call(
        body,
        grid=(NB,),
        in_specs=[
            pl.BlockSpec(memory_space=pltpu.SMEM),
            pl.BlockSpec((TB, D), lambda i: (i, 0)),
            pl.BlockSpec((1, 1, TB), lambda i: (i, 0, 0)),
            pl.BlockSpec((NW * PP, D), lambda i: (0, 0)),
            pl.BlockSpec((NW, PP), lambda i: (0, 0)),
            pl.BlockSpec((NW, PP), lambda i: (0, 0)),
        ],
        out_specs=pl.BlockSpec(memory_space=pltpu.SMEM),
        out_shape=jax.ShapeDtypeStruct((1,), jnp.float32),
        scratch_shapes=[
            pltpu.VMEM((PP, D), jnp.bfloat16),
            pltpu.VMEM((PP, 1), jnp.float32),
            pltpu.SMEM((8,), jnp.float32),
        ],
        interpret=interpret,
    )


def kernel(features, scores, phn_ids, lambda_d=0.5, lambda_t=0.1, margin=1.0):
    B, T, D = features.shape
    N = B * T
    feat2 = features.reshape(N, D)
    featf = features.reshape(-1)
    phn = phn_ids.reshape(-1).astype(jnp.int32)
    sc = scores.reshape(-1).astype(jnp.float32)

    cent_parts, cnt_parts, hc_parts = _make_sc_segment_sums(N, D, 256)(
        featf, phn, sc)
    cent2 = cent_parts.reshape(NW * PP, D)

    NB, TB = 8, N // 8
    lam = jnp.stack([jnp.asarray(lambda_d, jnp.float32),
                     jnp.asarray(lambda_t, jnp.float32),
                     jnp.asarray(margin, jnp.float32)])
    packed = phn | (sc.astype(jnp.int32) << 7)
    out = _make_tc_loss(N, D, NB, TB)(
        lam, feat2, packed.reshape(NB, 1, TB),
        cent2, cnt_parts, hc_parts)
    return out[0]
